# baseline (speedup 1.0000x reference)
import jax
import jax.numpy as jnp
from jax.experimental import pallas as pl
from jax.experimental.pallas import tpu as pltpu

_NSAMPLES = 2048
_K = 4


def _fps_kernel(x_ref, y_ref, z_ref, idx_ref, sx_ref, sy_ref, sz_ref,
                dists_ref):
    n, p = x_ref.shape
    s = idx_ref.shape[1]
    ch = p // 128
    per = ch // _K
    dists_ref[...] = jnp.full((n, p), jnp.inf, dtype=jnp.float32)
    lane = jax.lax.broadcasted_iota(jnp.int32, (n, 128), 1)

    def body(i, carry):
        bi, bx, by, bz, fx, fy, fz = carry

        def chunk_pass(c):
            sl = slice(c * 128, (c + 1) * 128)
            xv = x_ref[:, sl]
            yv = y_ref[:, sl]
            zv = z_ref[:, sl]
            dx = xv - fx
            dy = yv - fy
            dz = zv - fz
            d = dx * dx + dy * dy + dz * dz
            nd = jnp.minimum(dists_ref[:, sl], d)
            dists_ref[:, sl] = nd
            return nd, xv, yv, zv

        groups = []
        for g in range(_K):
            c0 = g * per
            nd, xacc, yacc, zacc = chunk_pass(c0)
            macc = nd
            cacc = jnp.full((n, 128), c0, jnp.int32)
            for c in range(c0 + 1, c0 + per):
                nd, xv, yv, zv = chunk_pass(c)
                better = nd > macc
                macc = jnp.maximum(nd, macc)
                cacc = jnp.where(better, c, cacc)
                xacc = jnp.where(better, xv, xacc)
                yacc = jnp.where(better, yv, yacc)
                zacc = jnp.where(better, zv, zacc)
            mg = jnp.max(macc, axis=1, keepdims=True)
            groups.append((mg, macc, cacc, xacc, yacc, zacc))

        m = groups[0][0]
        for g in range(1, _K):
            m = jnp.maximum(m, groups[g][0])
        mg, macc, cacc, xacc, yacc, zacc = groups[_K - 1]
        for g in range(_K - 2, -1, -1):
            gmg, gm, gc, gx, gy, gz = groups[g]
            hit = gmg == m
            macc = jnp.where(hit, gm, macc)
            cacc = jnp.where(hit, gc, cacc)
            xacc = jnp.where(hit, gx, xacc)
            yacc = jnp.where(hit, gy, yacc)
            zacc = jnp.where(hit, gz, zacc)

        eqm = macc == m
        flat = cacc * 128 + lane
        nf = jnp.min(jnp.where(eqm, flat, p), axis=1, keepdims=True)
        pick = flat == nf
        zf = jnp.zeros((n, 128), jnp.float32)
        nfx = jnp.sum(jnp.where(pick, xacc, zf), axis=1, keepdims=True)
        nfy = jnp.sum(jnp.where(pick, yacc, zf), axis=1, keepdims=True)
        nfz = jnp.sum(jnp.where(pick, zacc, zf), axis=1, keepdims=True)

        j = i + 1
        jlane = jax.lax.rem(j, 128)
        lmask = lane == jlane
        nbi = jnp.where(lmask, nf, bi)
        nbx = jnp.where(lmask, nfx, bx)
        nby = jnp.where(lmask, nfy, by)
        nbz = jnp.where(lmask, nfz, bz)

        @pl.when(jlane == 127)
        def _flush():
            base = pl.multiple_of(i - 126, 128)
            idx_ref[:, pl.ds(base, 128)] = nbi
            sx_ref[:, pl.ds(base, 128)] = nbx
            sy_ref[:, pl.ds(base, 128)] = nby
            sz_ref[:, pl.ds(base, 128)] = nbz

        return (nbi, nbx, nby, nbz, nfx, nfy, nfz)

    zi = jnp.zeros((n, 128), jnp.int32)
    lane0 = lane == 0
    zf128 = jnp.zeros((n, 128), jnp.float32)
    bx0 = jnp.where(lane0, x_ref[:, 0:1], zf128)
    by0 = jnp.where(lane0, y_ref[:, 0:1], zf128)
    bz0 = jnp.where(lane0, z_ref[:, 0:1], zf128)
    fx0 = x_ref[:, 0:1]
    fy0 = y_ref[:, 0:1]
    fz0 = z_ref[:, 0:1]
    jax.lax.fori_loop(0, s, body, (zi, bx0, by0, bz0, fx0, fy0, fz0))


def kernel(points):
    n, p, _ = points.shape
    s = _NSAMPLES
    pts = jnp.transpose(points, (2, 0, 1))
    x, y, z = pts[0], pts[1], pts[2]

    idx, sx, sy, sz = pl.pallas_call(
        _fps_kernel,
        out_shape=(
            jax.ShapeDtypeStruct((n, s), jnp.int32),
            jax.ShapeDtypeStruct((n, s), jnp.float32),
            jax.ShapeDtypeStruct((n, s), jnp.float32),
            jax.ShapeDtypeStruct((n, s), jnp.float32),
        ),
        scratch_shapes=[
            pltpu.VMEM((n, p), jnp.float32),
        ],
    )(x, y, z)

    sampled = jnp.stack([sx, sy, sz], axis=-1)
    return idx, sampled


# speedup vs baseline: 1.0105x; 1.0105x over previous
import jax
import jax.numpy as jnp
from jax.experimental import pallas as pl
from jax.experimental.pallas import tpu as pltpu

_NSAMPLES = 2048
_K = 4


def _fps_kernel(x_ref, y_ref, z_ref, idx_ref, sx_ref, sy_ref, sz_ref,
                dists_ref):
    n, p = x_ref.shape
    s = idx_ref.shape[1]
    ch = p // 128
    per = ch // _K
    dists_ref[...] = jnp.full((n, p), jnp.inf, dtype=jnp.float32)
    lane = jax.lax.broadcasted_iota(jnp.int32, (n, 128), 1)

    def body(i, carry):
        bi, bx, by, bz, fx, fy, fz = carry

        def chunk_pass(c):
            sl = slice(c * 128, (c + 1) * 128)
            xv = x_ref[:, sl]
            yv = y_ref[:, sl]
            zv = z_ref[:, sl]
            dx = xv - fx
            dy = yv - fy
            dz = zv - fz
            d = dx * dx + dy * dy + dz * dz
            nd = jnp.minimum(dists_ref[:, sl], d)
            dists_ref[:, sl] = nd
            return nd, xv, yv, zv

        groups = []
        for g in range(_K):
            c0 = g * per
            nd, xacc, yacc, zacc = chunk_pass(c0)
            macc = nd
            cacc = jnp.full((n, 128), c0, jnp.int32)
            for c in range(c0 + 1, c0 + per):
                nd, xv, yv, zv = chunk_pass(c)
                better = nd > macc
                macc = jnp.maximum(nd, macc)
                cacc = jnp.where(better, c, cacc)
                xacc = jnp.where(better, xv, xacc)
                yacc = jnp.where(better, yv, yacc)
                zacc = jnp.where(better, zv, zacc)
            groups.append((macc, cacc, xacc, yacc, zacc))

        macc, cacc, xacc, yacc, zacc = groups[0]
        for g in range(1, _K):
            gm, gc, gx, gy, gz = groups[g]
            better = gm > macc
            macc = jnp.maximum(gm, macc)
            cacc = jnp.where(better, gc, cacc)
            xacc = jnp.where(better, gx, xacc)
            yacc = jnp.where(better, gy, yacc)
            zacc = jnp.where(better, gz, zacc)

        m = jnp.max(macc, axis=1, keepdims=True)
        eqm = macc == m
        flat = cacc * 128 + lane
        nf = jnp.min(jnp.where(eqm, flat, p), axis=1, keepdims=True)
        pick = flat == nf
        zf = jnp.zeros((n, 128), jnp.float32)
        nfx = jnp.sum(jnp.where(pick, xacc, zf), axis=1, keepdims=True)
        nfy = jnp.sum(jnp.where(pick, yacc, zf), axis=1, keepdims=True)
        nfz = jnp.sum(jnp.where(pick, zacc, zf), axis=1, keepdims=True)

        j = i + 1
        jlane = jax.lax.rem(j, 128)
        lmask = lane == jlane
        nbi = jnp.where(lmask, nf, bi)
        nbx = jnp.where(lmask, nfx, bx)
        nby = jnp.where(lmask, nfy, by)
        nbz = jnp.where(lmask, nfz, bz)

        @pl.when(jlane == 127)
        def _flush():
            base = pl.multiple_of(i - 126, 128)
            idx_ref[:, pl.ds(base, 128)] = nbi
            sx_ref[:, pl.ds(base, 128)] = nbx
            sy_ref[:, pl.ds(base, 128)] = nby
            sz_ref[:, pl.ds(base, 128)] = nbz

        return (nbi, nbx, nby, nbz, nfx, nfy, nfz)

    zi = jnp.zeros((n, 128), jnp.int32)
    lane0 = lane == 0
    zf128 = jnp.zeros((n, 128), jnp.float32)
    bx0 = jnp.where(lane0, x_ref[:, 0:1], zf128)
    by0 = jnp.where(lane0, y_ref[:, 0:1], zf128)
    bz0 = jnp.where(lane0, z_ref[:, 0:1], zf128)
    fx0 = x_ref[:, 0:1]
    fy0 = y_ref[:, 0:1]
    fz0 = z_ref[:, 0:1]
    jax.lax.fori_loop(0, s, body, (zi, bx0, by0, bz0, fx0, fy0, fz0))


def kernel(points):
    n, p, _ = points.shape
    s = _NSAMPLES
    pts = jnp.transpose(points, (2, 0, 1))
    x, y, z = pts[0], pts[1], pts[2]

    idx, sx, sy, sz = pl.pallas_call(
        _fps_kernel,
        out_shape=(
            jax.ShapeDtypeStruct((n, s), jnp.int32),
            jax.ShapeDtypeStruct((n, s), jnp.float32),
            jax.ShapeDtypeStruct((n, s), jnp.float32),
            jax.ShapeDtypeStruct((n, s), jnp.float32),
        ),
        scratch_shapes=[
            pltpu.VMEM((n, p), jnp.float32),
        ],
    )(x, y, z)

    sampled = jnp.stack([sx, sy, sz], axis=-1)
    return idx, sampled


# speedup vs baseline: 1.0514x; 1.0405x over previous
import jax
import jax.numpy as jnp
from jax.experimental import pallas as pl
from jax.experimental.pallas import tpu as pltpu

_NSAMPLES = 2048
_K = 2


def _fps_kernel(x_ref, y_ref, z_ref, idx_ref, sx_ref, sy_ref, sz_ref,
                dists_ref):
    n, p = x_ref.shape
    s = idx_ref.shape[1]
    ch = p // 128
    per = ch // _K
    dists_ref[...] = jnp.full((n, p), jnp.inf, dtype=jnp.float32)
    lane = jax.lax.broadcasted_iota(jnp.int32, (n, 128), 1)

    def body(i, carry):
        bi, bx, by, bz, fx, fy, fz = carry

        def chunk_pass(c):
            sl = slice(c * 128, (c + 1) * 128)
            xv = x_ref[:, sl]
            yv = y_ref[:, sl]
            zv = z_ref[:, sl]
            dx = xv - fx
            dy = yv - fy
            dz = zv - fz
            d = dx * dx + dy * dy + dz * dz
            nd = jnp.minimum(dists_ref[:, sl], d)
            dists_ref[:, sl] = nd
            return nd, xv, yv, zv

        groups = []
        for g in range(_K):
            c0 = g * per
            nd, xacc, yacc, zacc = chunk_pass(c0)
            macc = nd
            cacc = jnp.full((n, 128), c0, jnp.int32)
            for c in range(c0 + 1, c0 + per):
                nd, xv, yv, zv = chunk_pass(c)
                better = nd > macc
                macc = jnp.maximum(nd, macc)
                cacc = jnp.where(better, c, cacc)
                xacc = jnp.where(better, xv, xacc)
                yacc = jnp.where(better, yv, yacc)
                zacc = jnp.where(better, zv, zacc)
            groups.append((macc, cacc, xacc, yacc, zacc))

        macc, cacc, xacc, yacc, zacc = groups[0]
        for g in range(1, _K):
            gm, gc, gx, gy, gz = groups[g]
            better = gm > macc
            macc = jnp.maximum(gm, macc)
            cacc = jnp.where(better, gc, cacc)
            xacc = jnp.where(better, gx, xacc)
            yacc = jnp.where(better, gy, yacc)
            zacc = jnp.where(better, gz, zacc)

        m = jnp.max(macc, axis=1, keepdims=True)
        eqm = macc == m
        flat = cacc * 128 + lane
        nf = jnp.min(jnp.where(eqm, flat, p), axis=1, keepdims=True)
        pick = flat == nf
        zf = jnp.zeros((n, 128), jnp.float32)
        nfx = jnp.sum(jnp.where(pick, xacc, zf), axis=1, keepdims=True)
        nfy = jnp.sum(jnp.where(pick, yacc, zf), axis=1, keepdims=True)
        nfz = jnp.sum(jnp.where(pick, zacc, zf), axis=1, keepdims=True)

        j = i + 1
        jlane = jax.lax.rem(j, 128)
        lmask = lane == jlane
        nbi = jnp.where(lmask, nf, bi)
        nbx = jnp.where(lmask, nfx, bx)
        nby = jnp.where(lmask, nfy, by)
        nbz = jnp.where(lmask, nfz, bz)

        @pl.when(jlane == 127)
        def _flush():
            base = pl.multiple_of(i - 126, 128)
            idx_ref[:, pl.ds(base, 128)] = nbi
            sx_ref[:, pl.ds(base, 128)] = nbx
            sy_ref[:, pl.ds(base, 128)] = nby
            sz_ref[:, pl.ds(base, 128)] = nbz

        return (nbi, nbx, nby, nbz, nfx, nfy, nfz)

    zi = jnp.zeros((n, 128), jnp.int32)
    lane0 = lane == 0
    zf128 = jnp.zeros((n, 128), jnp.float32)
    bx0 = jnp.where(lane0, x_ref[:, 0:1], zf128)
    by0 = jnp.where(lane0, y_ref[:, 0:1], zf128)
    bz0 = jnp.where(lane0, z_ref[:, 0:1], zf128)
    fx0 = x_ref[:, 0:1]
    fy0 = y_ref[:, 0:1]
    fz0 = z_ref[:, 0:1]
    jax.lax.fori_loop(0, s, body, (zi, bx0, by0, bz0, fx0, fy0, fz0))


def kernel(points):
    n, p, _ = points.shape
    s = _NSAMPLES
    pts = jnp.transpose(points, (2, 0, 1))
    x, y, z = pts[0], pts[1], pts[2]

    idx, sx, sy, sz = pl.pallas_call(
        _fps_kernel,
        out_shape=(
            jax.ShapeDtypeStruct((n, s), jnp.int32),
            jax.ShapeDtypeStruct((n, s), jnp.float32),
            jax.ShapeDtypeStruct((n, s), jnp.float32),
            jax.ShapeDtypeStruct((n, s), jnp.float32),
        ),
        scratch_shapes=[
            pltpu.VMEM((n, p), jnp.float32),
        ],
    )(x, y, z)

    sampled = jnp.stack([sx, sy, sz], axis=-1)
    return idx, sampled
